# baseline (speedup 1.0000x reference)
import jax
import jax.numpy as jnp
from jax.experimental import pallas as pl

N1 = 8600
N2 = 8000


def _edgeconv(edges, x, p, n):
    tw, tb, pw, pb = p
    src = edges[0]
    dst = edges[1]
    xj = jnp.take(x, src, axis=0)
    xi = jnp.take(x, dst, axis=0)
    e = (xj - xi) @ tw.T + tb + xi @ pw.T + pb
    h = jax.ops.segment_max(e, dst, num_segments=n)
    return jnp.where(jnp.isfinite(h), h, 0.0)


def _relu_kernel(x_ref, o_ref):
    o_ref[...] = jnp.maximum(x_ref[...], 0.0)


def kernel(graph, graph_vox, inputs, bs, koor, params):
    zeros = jnp.zeros((1, 8000, 51), dtype=inputs.dtype)
    new_inputs = jnp.concatenate(
        (jnp.concatenate((inputs, zeros), axis=1), jnp.reshape(koor, (1, 8600, 3))), axis=2
    )
    feat = jnp.reshape(new_inputs, (8600, 54)).astype(jnp.float32)
    for i in range(8):
        feat = _edgeconv(graph, feat, params[i], N1)
    feat = feat[600:, :]
    for i in range(8, 23):
        feat = _edgeconv(graph_vox, feat, params[i], N2)
    out = feat.reshape(-1)
    padded = jnp.pad(out, (0, 192)).reshape(64, 128)
    r = pl.pallas_call(
        _relu_kernel,
        out_shape=jax.ShapeDtypeStruct((64, 128), jnp.float32),
    )(padded)
    return r.reshape(-1)[:8000]


# speedup vs baseline: 2.4776x; 2.4776x over previous
import functools

import jax
import jax.numpy as jnp
from jax import lax
from jax.experimental import pallas as pl
from jax.experimental.pallas import tpu as pltpu
from jax.experimental.pallas import tpu_sc as plsc

N1, N2 = 8600, 8000
NP1, NP2 = 8704, 8192
NC, NS = 2, 16
NW = NC * NS

_MESH = dict(core_axis_name="c", subcore_axis_name="s")


def _epad(e, ch):
    epw = -(-e // (NW * ch)) * ch
    return NW * epw


@functools.lru_cache(None)
def _mme(epad, kp, dp):
    blk = 2048
    assert epad % blk == 0

    def body(d_ref, w_ref, o_ref):
        o_ref[...] = jnp.dot(
            d_ref[...], w_ref[...], preferred_element_type=jnp.float32
        )

    return pl.pallas_call(
        body,
        grid=(epad // blk,),
        in_specs=[
            pl.BlockSpec((blk, kp), lambda i: (i, 0)),
            pl.BlockSpec((kp, dp), lambda i: (0, 0)),
        ],
        out_specs=pl.BlockSpec((blk, dp), lambda i: (i, 0)),
        out_shape=jax.ShapeDtypeStruct((epad, dp), jnp.float32),
    )


@functools.lru_cache(None)
def _mmv(np_, kp, dp):
    def body(x_ref, w_ref, o_ref):
        o_ref[...] = jnp.dot(
            x_ref[...], w_ref[...], preferred_element_type=jnp.float32
        )

    return pl.pallas_call(
        body,
        out_shape=jax.ShapeDtypeStruct((np_, dp), jnp.float32),
    )


@functools.lru_cache(None)
def _dsub(np_, dp, epad, ch):
    epw = epad // NW
    ncha = epw // ch
    mesh = plsc.VectorSubcoreMesh(**_MESH)

    def body(x_hbm, xn_hbm, srcs_hbm, dsts_hbm, d_hbm,
             idxs_v, idxd_v, rows_v, sem):
        wid = lax.axis_index("s") * NC + lax.axis_index("c")
        e0 = wid * epw
        for c in range(ncha):
            cb = e0 + c * ch
            pltpu.sync_copy(srcs_hbm.at[pl.ds(cb, ch)], idxs_v)
            pltpu.sync_copy(dsts_hbm.at[pl.ds(cb, ch)], idxd_v)
            pltpu.async_copy(x_hbm.at[idxs_v], rows_v, sem).wait()
            pltpu.async_copy(xn_hbm.at[idxd_v], rows_v, sem, add=True).wait()
            pltpu.sync_copy(rows_v, d_hbm.at[pl.ds(cb, ch)])

    return pl.kernel(
        body,
        out_type=jax.ShapeDtypeStruct((epad, dp), jnp.float32),
        mesh=mesh,
        compiler_params=pltpu.CompilerParams(use_tc_tiling_on_sc=False),
        scratch_types=[
            pltpu.VMEM((ch,), jnp.int32),
            pltpu.VMEM((ch,), jnp.int32),
            pltpu.VMEM((ch, dp), jnp.float32),
            pltpu.SemaphoreType.DMA,
        ],
    )


def _tabw(e, ch):
    return (((e + 8 + ch - 1) // ch + 2) + 15) // 16 * 16 + 16


@functools.lru_cache(None)
def _segmax(np_, dp, e, ch, relu):
    npw = np_ // NW
    offn = npw + 32
    nvec = dp // 16
    tabw = _tabw(e, ch)
    mesh = plsc.VectorSubcoreMesh(**_MESH)

    def body(eth_hbm, v_hbm, tb_hbm, pb_hbm, offs_hbm, tab_hbm,
             out_hbm, outn_hbm,
             offs_v, tab_v, rows_v, vloc, oloc, olocn, tb_v, pb_v):
        wid = lax.axis_index("s") * NC + lax.axis_index("c")
        n0 = wid * npw
        pltpu.sync_copy(offs_hbm.at[pl.ds(n0, offn)], offs_v)
        pltpu.sync_copy(tab_hbm.at[wid], tab_v)
        pltpu.sync_copy(v_hbm.at[pl.ds(n0, npw)], vloc)
        pltpu.sync_copy(tb_hbm, tb_v)
        pltpu.sync_copy(pb_hbm, pb_v)

        def geto(i):
            return offs_v[pl.ds(i, 16)][0]

        def gett(i):
            return tab_v[pl.ds(i, 16)][0]

        e_lo = geto(0)
        e_hi = geto(npw)
        cstart = (e_lo // 8) * 8
        nch = (e_hi - cstart + (ch - 1)) // ch

        neg = jnp.full((16,), -jnp.inf, jnp.float32)
        zero = jnp.zeros((16,), jnp.float32)

        def run_edges(s, t, acc):
            def eb(j, a):
                return tuple(
                    jnp.maximum(a[k], rows_v[j, pl.ds(k * 16, 16)])
                    for k in range(nvec)
                )
            return lax.fori_loop(s, t, eb, acc)

        def finalize(n, deg, acc):
            for k in range(nvec):
                sl = pl.ds(k * 16, 16)
                val = ((acc[k] + tb_v[sl]) + vloc[n, sl]) + pb_v[sl]
                if relu:
                    val = jnp.maximum(val, 0.0)
                val = jnp.where(deg > 0, val, zero)
                oloc[n, sl] = val
                olocn[n, sl] = -val

        head = jnp.where(nch > 0, gett(0), jnp.int32(npw))

        def z_body(n, carry):
            for k in range(nvec):
                oloc[n, pl.ds(k * 16, 16)] = zero
                olocn[n, pl.ds(k * 16, 16)] = zero
            return carry

        lax.fori_loop(0, head, z_body, 0)

        def chunk_body(c, acc):
            cbase = pl.multiple_of(cstart + c * ch, 8)
            climit = cbase + ch
            pltpu.sync_copy(eth_hbm.at[pl.ds(cbase, ch)], rows_v)

            def node_body(nn, acc):
                s_full = geto(nn)
                t_full = geto(nn + 1)
                s = jnp.maximum(s_full, cbase)
                acc = run_edges(s - cbase, t_full - cbase, acc)
                finalize(nn, t_full - s_full, acc)
                return (neg,) * nvec

            acc = lax.fori_loop(gett(c), gett(c + 1), node_body, acc)
            s = jnp.minimum(jnp.maximum(geto(gett(c + 1)), cbase), climit)
            acc = run_edges(s - cbase, climit - cbase, acc)
            return acc

        lax.fori_loop(0, nch, chunk_body, (neg,) * nvec)
        pltpu.sync_copy(oloc, out_hbm.at[pl.ds(n0, npw)])
        pltpu.sync_copy(olocn, outn_hbm.at[pl.ds(n0, npw)])

    return pl.kernel(
        body,
        out_type=(
            jax.ShapeDtypeStruct((np_, dp), jnp.float32),
            jax.ShapeDtypeStruct((np_, dp), jnp.float32),
        ),
        mesh=mesh,
        compiler_params=pltpu.CompilerParams(use_tc_tiling_on_sc=False),
        scratch_types=[
            pltpu.VMEM((offn,), jnp.int32),
            pltpu.VMEM((tabw,), jnp.int32),
            pltpu.VMEM((ch, dp), jnp.float32),
            pltpu.VMEM((npw, dp), jnp.float32),
            pltpu.VMEM((npw, dp), jnp.float32),
            pltpu.VMEM((npw, dp), jnp.float32),
            pltpu.VMEM((dp,), jnp.float32),
            pltpu.VMEM((dp,), jnp.float32),
        ],
    )


def _csr(edges, np_, epad):
    src = edges[0].astype(jnp.int32)
    dst = edges[1].astype(jnp.int32)
    order = jnp.argsort(dst)
    dst_s = jnp.take(dst, order)
    src_s = jnp.take(src, order)
    offs = jnp.searchsorted(
        dst_s, jnp.arange(np_ + 40, dtype=jnp.int32)
    ).astype(jnp.int32)
    pad = epad - src.shape[0]
    srcs = jnp.concatenate([src_s, jnp.zeros((pad,), jnp.int32)])
    dsts = jnp.concatenate([dst_s, jnp.zeros((pad,), jnp.int32)])
    return srcs, dsts, offs


def _tab(offs, np_, e, ch):
    npw = np_ // NW
    tabw = _tabw(e, ch)
    n0s = jnp.arange(NW, dtype=jnp.int32) * npw
    cstart = (offs[n0s] // 8) * 8
    ends = offs[1 : np_ + 1].reshape(NW, npw)
    bounds = cstart[:, None] + jnp.arange(tabw, dtype=jnp.int32)[None, :] * ch
    tab = jax.vmap(
        lambda en, bd: jnp.searchsorted(en, bd, side="right")
    )(ends, bounds)
    return tab.astype(jnp.int32)


def _prep(p, kp, dp):
    tw, tb, pw, pb = p
    w1 = jnp.zeros((kp, dp), jnp.float32).at[: tw.shape[1], : tw.shape[0]].set(tw.T)
    w2 = jnp.zeros((kp, dp), jnp.float32).at[: pw.shape[1], : pw.shape[0]].set(pw.T)
    tbp = jnp.zeros((dp,), jnp.float32).at[: tb.shape[0]].set(tb)
    pbp = jnp.zeros((dp,), jnp.float32).at[: pb.shape[0]].set(pb)
    return w1, w2, tbp, pbp


_DIMS_PAD = [(64, 64)] * 12 + [(64, 32)] + [(32, 32)] * 9 + [(32, 16)]


def kernel(graph, graph_vox, inputs, bs, koor, params):
    x = jnp.zeros((NP1, 64), jnp.float32)
    x = x.at[:600, :51].set(inputs[0].astype(jnp.float32))
    x = x.at[:N1, 51:54].set(koor[0].astype(jnp.float32))
    xn = -x

    e1 = graph.shape[1]
    e2 = graph_vox.shape[1]
    ep1 = _epad(e1, 512)
    ep2 = _epad(e2, 1024)
    srcs1, dsts1, offs1 = _csr(graph, NP1, ep1)
    srcs2, dsts2, offs2 = _csr(graph_vox, NP2, ep2)
    tab1 = _tab(offs1, NP1, e1, 512)
    tab2a = _tab(offs2, NP2, e2, 512)
    tab2b = _tab(offs2, NP2, e2, 1024)

    for i in range(8):
        kp, dp = _DIMS_PAD[i]
        w1, w2, tb, pb = _prep(params[i], kp, dp)
        d = _dsub(NP1, kp, ep1, 512)(x, xn, srcs1, dsts1)
        eth = _mme(ep1, kp, dp)(d, w1)
        v = _mmv(NP1, kp, dp)(x, w2)
        x, xn = _segmax(NP1, dp, e1, 512, False)(eth, v, tb, pb, offs1, tab1)

    x2 = jnp.zeros((NP2, 64), jnp.float32).at[:N2].set(x[600:N1])
    xn2 = jnp.zeros((NP2, 64), jnp.float32).at[:N2].set(xn[600:N1])
    x, xn = x2, xn2

    for i in range(8, 23):
        kp, dp = _DIMS_PAD[i]
        ch = 512 if dp == 64 else 1024
        tab = tab2a if ch == 512 else tab2b
        w1, w2, tb, pb = _prep(params[i], kp, dp)
        d = _dsub(NP2, kp, ep2, ch)(x, xn, srcs2, dsts2)
        eth = _mme(ep2, kp, dp)(d, w1)
        v = _mmv(NP2, kp, dp)(x, w2)
        x, xn = _segmax(NP2, dp, e2, ch, i == 22)(eth, v, tb, pb, offs2, tab)

    return x[:N2, 0]
